# baseline (speedup 1.0000x reference)
import functools

import jax
import jax.numpy as jnp
from jax import lax
from jax.experimental import pallas as pl
from jax.experimental.pallas import tpu as pltpu
from jax.experimental.pallas import tpu_sc as plsc

_B = 16384
_D = 64
_NC = 2
_NS = 16
_NW = _NC * _NS
_BPW = _B // _NW
_CHUNK = 128
_NCHUNK = _BPW // _CHUNK
_LANES = 16


def _sc_partials(labels2, features3, centers):
    mesh = plsc.VectorSubcoreMesh(core_axis_name="c", subcore_axis_name="s")

    @functools.partial(
        pl.kernel,
        mesh=mesh,
        out_type=jax.ShapeDtypeStruct((_NW, _LANES), jnp.float32),
        scratch_types=[
            pltpu.VMEM((_NCHUNK, _CHUNK), jnp.int32),
            pltpu.VMEM((_NCHUNK, _CHUNK, _D), jnp.float32),
            pltpu.VMEM((_NCHUNK, _CHUNK, _D), jnp.float32),
            pltpu.VMEM((_LANES,), jnp.float32),
            pltpu.SemaphoreType.DMA,
            pltpu.SemaphoreType.DMA,
        ],
        compiler_params=pltpu.CompilerParams(use_tc_tiling_on_sc=False),
    )
    def k(labels_hbm, feats_hbm, cen_hbm, out_hbm, idx_v, cen_v, feat_v,
          acc_v, gsem, fsem):
        wid = lax.axis_index("s") * _NC + lax.axis_index("c")
        row0 = wid * _NCHUNK

        fcopy = pltpu.async_copy(
            feats_hbm.at[pl.ds(row0, _NCHUNK)], feat_v, fsem)
        pltpu.sync_copy(labels_hbm.at[pl.ds(row0, _NCHUNK)], idx_v)
        copies = [
            pltpu.async_copy(cen_hbm.at[idx_v.at[c]], cen_v.at[c], gsem)
            for c in range(_NCHUNK)
        ]
        fcopy.wait()
        for c in copies:
            c.wait()

        def body(j, acc):
            for c in range(_NCHUNK):
                for t in range(_D // _LANES):
                    d = (cen_v[c, j, pl.ds(t * _LANES, _LANES)]
                         - feat_v[c, j, pl.ds(t * _LANES, _LANES)])
                    acc = acc + d * d
            return acc

        acc = lax.fori_loop(0, _CHUNK, body, jnp.zeros((_LANES,), jnp.float32))
        acc_v[...] = acc
        pltpu.sync_copy(acc_v, out_hbm.at[wid])

    return k(labels2, features3, centers)


def _finish(p_ref, o_ref):
    o_ref[0] = jnp.sum(p_ref[...]) * (0.5 / _B)


def kernel(features, labels, centers):
    labels2 = labels.astype(jnp.int32).reshape(_B // _CHUNK, _CHUNK)
    features3 = features.reshape(_B // _CHUNK, _CHUNK, _D)
    partials = _sc_partials(labels2, features3, centers)
    loss = pl.pallas_call(
        _finish,
        out_shape=jax.ShapeDtypeStruct((1,), jnp.float32),
        out_specs=pl.BlockSpec(memory_space=pltpu.SMEM),
    )(partials)
    return loss[0]


# speedup vs baseline: 1.6446x; 1.6446x over previous
import functools

import jax
import jax.numpy as jnp
from jax import lax
from jax.experimental import pallas as pl
from jax.experimental.pallas import tpu as pltpu
from jax.experimental.pallas import tpu_sc as plsc

_B = 16384
_D = 64
_NC = 2
_NS = 16
_NW = _NC * _NS
_BPW = _B // _NW
_GC = 64
_NCH = _BPW // _GC
_LANES = 16


def _sc_partials(labels1, features, centers):
    mesh = plsc.VectorSubcoreMesh(core_axis_name="c", subcore_axis_name="s")

    @functools.partial(
        pl.kernel,
        mesh=mesh,
        out_type=jax.ShapeDtypeStruct((_NW, _LANES), jnp.float32),
        scratch_types=[
            pltpu.VMEM((_BPW + _LANES,), jnp.int32),
            pltpu.VMEM((2, _GC, 1, _D), jnp.float32),
            pltpu.VMEM((_BPW, _D), jnp.float32),
            pltpu.VMEM((_LANES,), jnp.float32),
            pltpu.SemaphoreType.DMA,
            pltpu.SemaphoreType.DMA,
            pltpu.SemaphoreType.DMA,
        ],
        compiler_params=pltpu.CompilerParams(use_tc_tiling_on_sc=True),
    )
    def k(labels_hbm, feats_hbm, cen_hbm, out_hbm, lab_v,
          g_v, feat_v, acc_v, gsem0, gsem1, fsem):
        wid = lax.axis_index("s") * _NC + lax.axis_index("c")
        base = wid * _BPW

        fcopy = pltpu.async_copy(
            feats_hbm.at[pl.ds(base, _BPW)], feat_v, fsem)
        pltpu.sync_copy(labels_hbm.at[pl.ds(base, _BPW)],
                        lab_v.at[pl.ds(0, _BPW)])

        gsems = [gsem0, gsem1]

        def gather(c):
            hs = []
            for j in range(_GC):
                l = lab_v[pl.ds(c * _GC + j, _LANES)][0]
                hs.append(pltpu.async_copy(
                    cen_hbm.at[pl.ds(l, 1)], g_v.at[c % 2, j], gsems[c % 2]))
            return hs

        handles = [gather(0), gather(1)]
        fcopy.wait()

        def chunk_body(c, j, acc):
            jj = c * _GC + j
            for t in range(_D // _LANES):
                d = (g_v[c % 2, j, 0, pl.ds(t * _LANES, _LANES)]
                     - feat_v[jj, pl.ds(t * _LANES, _LANES)])
                acc = acc + d * d
            return acc

        acc = jnp.zeros((_LANES,), jnp.float32)
        for c in range(_NCH):
            for h in handles[c % 2]:
                h.wait()
            acc = lax.fori_loop(
                0, _GC, functools.partial(chunk_body, c), acc)
            if c + 2 < _NCH:
                handles[c % 2] = gather(c + 2)

        acc_v[...] = acc
        pltpu.sync_copy(acc_v, out_hbm.at[wid])

    return k(labels1, features, centers)


def _finish(p_ref, o_ref):
    o_ref[0] = jnp.sum(p_ref[...]) * (0.5 / _B)


def kernel(features, labels, centers):
    labels1 = labels.astype(jnp.int32)
    partials = _sc_partials(labels1, features, centers)
    loss = pl.pallas_call(
        _finish,
        out_shape=jax.ShapeDtypeStruct((1,), jnp.float32),
        out_specs=pl.BlockSpec(memory_space=pltpu.SMEM),
    )(partials)
    return loss[0]
